# baseline (speedup 1.0000x reference)
import math

import jax
import jax.numpy as jnp
import numpy as np
from jax import lax
from jax.experimental import pallas as pl
from jax.experimental.pallas import tpu as pltpu
from jax.experimental.pallas import tpu_sc as plsc

VOCAB = 1000000
DIM = 64
SEQ = 200
BATCH = 4096

LANES = 16
CHUNK = 400
GSUB = 100
NGATH = CHUNK // GSUB


def _make_pe(dim: int, seq: int) -> np.ndarray:
    pe = np.zeros((seq, dim), dtype=np.float32)
    position = np.arange(0, seq, dtype=np.float32)[:, None]
    div_term = np.exp(
        np.arange(0, dim, 2, dtype=np.float32) * -(math.log(10000.0) / dim)
    )
    pe[:, 0::2] = np.sin(position * div_term)
    pe[:, 1::2] = np.cos(position * div_term)
    return pe


_PE = _make_pe(DIM, SEQ)


def kernel(x, embed_weight):
    n = BATCH * SEQ
    idx = x.reshape(-1).astype(jnp.int32).reshape(n // GSUB, GSUB)
    pe = jnp.asarray(_PE)

    mesh = plsc.VectorSubcoreMesh(core_axis_name="c", subcore_axis_name="s")
    nw = mesh.num_cores * mesh.num_subcores
    per_tile = n // nw
    n_chunks = per_tile // CHUNK
    idx_rows_per_chunk = CHUNK // GSUB

    def run(idx, table, pe):
        @pl.kernel(
            out_type=jax.ShapeDtypeStruct((n, DIM), jnp.float32),
            mesh=mesh,
            compiler_params=pltpu.CompilerParams(use_tc_tiling_on_sc=False),
            scratch_types=[
                pltpu.VMEM((idx_rows_per_chunk, GSUB), jnp.int32),
                pltpu.VMEM((CHUNK, DIM), jnp.float32),
                pltpu.VMEM((SEQ, DIM), jnp.float32),
                pltpu.SemaphoreType.DMA,
            ],
        )
        def k(idx_hbm, table_hbm, pe_hbm, out_hbm, idx_v, rows_v, pe_v, sem):
            wid = lax.axis_index("s") * mesh.num_cores + lax.axis_index("c")
            pltpu.sync_copy(pe_hbm, pe_v)
            tile_row0 = wid * (per_tile // GSUB)
            tile_base = wid * per_tile

            @pl.loop(0, n_chunks)
            def _(c):
                pltpu.sync_copy(
                    idx_hbm.at[pl.ds(tile_row0 + c * idx_rows_per_chunk,
                                     idx_rows_per_chunk)],
                    idx_v,
                )
                copies = [
                    pltpu.async_copy(
                        table_hbm.at[idx_v.at[g]],
                        rows_v.at[pl.ds(g * GSUB, GSUB)],
                        sem,
                    )
                    for g in range(NGATH)
                ]
                for cp in copies:
                    cp.wait()

                @pl.loop(0, SEQ)
                def _(r):
                    for col in range(0, DIM, LANES):
                        p = pe_v[r, col:col + LANES]
                        rows_v[r, col:col + LANES] = (
                            rows_v[r, col:col + LANES] * 8.0 + p
                        )
                        rows_v[r + SEQ, col:col + LANES] = (
                            rows_v[r + SEQ, col:col + LANES] * 8.0 + p
                        )

                pltpu.sync_copy(
                    rows_v, out_hbm.at[pl.ds(tile_base + c * CHUNK, CHUNK)]
                )

        return k(idx, table, pe)

    out = run(idx, embed_weight, pe)
    return out.reshape(BATCH, SEQ, DIM)


# speedup vs baseline: 1.1041x; 1.1041x over previous
import math

import jax
import jax.numpy as jnp
import numpy as np
from jax import lax
from jax.experimental import pallas as pl
from jax.experimental.pallas import tpu as pltpu
from jax.experimental.pallas import tpu_sc as plsc

VOCAB = 1000000
DIM = 64
SEQ = 200
BATCH = 4096

LANES = 16
CHUNK = 200
GSUB = 100
NG = CHUNK // GSUB
NBUF = 4
DEPTH = 2


def _make_pe(dim: int, seq: int) -> np.ndarray:
    pe = np.zeros((seq, dim), dtype=np.float32)
    position = np.arange(0, seq, dtype=np.float32)[:, None]
    div_term = np.exp(
        np.arange(0, dim, 2, dtype=np.float32) * -(math.log(10000.0) / dim)
    )
    pe[:, 0::2] = np.sin(position * div_term)
    pe[:, 1::2] = np.cos(position * div_term)
    return pe


_PE = _make_pe(DIM, SEQ)


def kernel(x, embed_weight):
    n = BATCH * SEQ
    idx = x.reshape(-1).astype(jnp.int32).reshape(n // GSUB, GSUB)
    pe = jnp.asarray(_PE)

    mesh = plsc.VectorSubcoreMesh(core_axis_name="c", subcore_axis_name="s")
    nw = mesh.num_cores * mesh.num_subcores
    per_tile = n // nw
    idx_rows = per_tile // GSUB
    n_chunks = per_tile // CHUNK
    n_rounds = n_chunks // NBUF

    @pl.kernel(
        out_type=jax.ShapeDtypeStruct((n, DIM), jnp.float32),
        mesh=mesh,
        compiler_params=pltpu.CompilerParams(use_tc_tiling_on_sc=False),
        scratch_types=(
            [pltpu.VMEM((idx_rows, GSUB), jnp.int32)]
            + [pltpu.VMEM((CHUNK, DIM), jnp.float32) for _ in range(NBUF)]
            + [pltpu.VMEM((SEQ, DIM), jnp.float32)]
            + [pltpu.SemaphoreType.DMA for _ in range(2 * NBUF)]
        ),
    )
    def k(idx_hbm, table_hbm, pe_hbm, out_hbm, idx_v, *rest):
        bufs = rest[:NBUF]
        pe_v = rest[NBUF]
        gsems = rest[NBUF + 1:NBUF + 1 + NBUF]
        wsems = rest[NBUF + 1 + NBUF:]

        wid = lax.axis_index("s") * mesh.num_cores + lax.axis_index("c")
        tile_base = wid * per_tile

        pltpu.sync_copy(pe_hbm, pe_v)
        pltpu.sync_copy(idx_hbm.at[pl.ds(wid * idx_rows, idx_rows)], idx_v)

        def fire_gather(ch, b):
            for g in range(NG):
                pltpu.async_copy(
                    table_hbm.at[idx_v.at[ch * NG + g]],
                    bufs[b].at[pl.ds(g * GSUB, GSUB)],
                    gsems[b],
                )

        def drain_gather(b):
            pltpu.make_async_copy(
                out_hbm.at[pl.ds(0, CHUNK)], bufs[b], gsems[b]
            ).wait()

        def fire_write(ch, b):
            pltpu.async_copy(
                bufs[b],
                out_hbm.at[pl.ds(tile_base + ch * CHUNK, CHUNK)],
                wsems[b],
            )

        def drain_write(b):
            pltpu.make_async_copy(
                bufs[b], out_hbm.at[pl.ds(0, CHUNK)], wsems[b]
            ).wait()

        def compute(b):
            buf = bufs[b]

            @pl.loop(0, CHUNK)
            def _(r):
                for col in range(0, DIM, LANES):
                    buf[r, col:col + LANES] = (
                        buf[r, col:col + LANES] * 8.0
                        + pe_v[r, col:col + LANES]
                    )

        def process(ch, b, reuse_write, fire_next):
            drain_gather(b)
            compute(b)
            fire_write(ch, b)
            if fire_next:
                bp = (b + DEPTH) % NBUF
                if reuse_write:
                    drain_write(bp)
                fire_gather(ch + DEPTH, bp)

        for ch in range(DEPTH):
            fire_gather(ch, ch % NBUF)

        for b in range(NBUF):
            process(b, b, reuse_write=(b + DEPTH >= NBUF), fire_next=True)

        @pl.loop(1, n_rounds - 1)
        def _(q):
            for b in range(NBUF):
                process(q * NBUF + b, b, reuse_write=True, fire_next=True)

        for b in range(NBUF):
            ch = (n_rounds - 1) * NBUF + b
            process(ch, b, reuse_write=True,
                    fire_next=(ch + DEPTH < n_chunks))

        for b in range(NBUF):
            drain_write(b)

    out = k(idx, embed_weight, pe)
    return out.reshape(BATCH, SEQ, DIM)


# speedup vs baseline: 1.7377x; 1.5738x over previous
import math

import jax
import jax.numpy as jnp
import numpy as np
from jax import lax
from jax.experimental import pallas as pl
from jax.experimental.pallas import tpu as pltpu
from jax.experimental.pallas import tpu_sc as plsc

VOCAB = 1000000
DIM = 64
SEQ = 200
BATCH = 4096

LANES = 16
PADW = 128
BT = 128


def _make_pe(dim: int, seq: int) -> np.ndarray:
    pe = np.zeros((seq, dim), dtype=np.float32)
    position = np.arange(0, seq, dtype=np.float32)[:, None]
    div_term = np.exp(
        np.arange(0, dim, 2, dtype=np.float32) * -(math.log(10000.0) / dim)
    )
    pe[:, 0::2] = np.sin(position * div_term)
    pe[:, 1::2] = np.cos(position * div_term)
    return pe


_PE = _make_pe(DIM, SEQ)


def kernel(x, embed_weight):
    xi = x.astype(jnp.int32).reshape(BATCH * SEQ)
    w128 = jnp.pad(embed_weight, ((0, 0), (0, PADW - DIM)))
    pe = jnp.asarray(_PE)

    mesh = plsc.VectorSubcoreMesh(core_axis_name="c", subcore_axis_name="s")
    nw = mesh.num_cores * mesh.num_subcores
    assert BATCH % BT == 0 and BATCH // BT == nw
    half = BT // 2 * SEQ

    @pl.kernel(
        out_type=jax.ShapeDtypeStruct((SEQ, DIM // 8, 32, 8, BT),
                                      jnp.float32),
        mesh=mesh,
        compiler_params=pltpu.CompilerParams(
            use_tc_tiling_on_sc=False, needs_layout_passes=False
        ),
        scratch_types=[
            pltpu.VMEM((half,), jnp.int32),
            pltpu.VMEM((SEQ, BT), jnp.int32),
            pltpu.VMEM((SEQ, DIM), jnp.float32),
            pltpu.VMEM((BT, PADW), jnp.float32),
            pltpu.VMEM((BT, PADW), jnp.float32),
            pltpu.VMEM((DIM // 8, 8, BT + 1), jnp.float32),
            pltpu.VMEM((DIM // 8, 8, BT + 1), jnp.float32),
            pltpu.SemaphoreType.DMA,
            pltpu.SemaphoreType.DMA,
            pltpu.SemaphoreType.DMA,
            pltpu.SemaphoreType.DMA,
        ],
    )
    def k(idx_hbm, table_hbm, pe_hbm, out_hbm, idx_h, idx_s, pe_v,
          rows0, rows1, st0, st1, g0, g1, w0, w1):
        rows = (rows0, rows1)
        stages = (st0, st1)
        gsems = (g0, g1)
        wsems = (w0, w1)

        wid = lax.axis_index("s") * mesh.num_cores + lax.axis_index("c")
        tile_idx0 = wid * (BT * SEQ)

        pltpu.sync_copy(pe_hbm, pe_v)

        pat = [(lax.iota(jnp.int32, LANES) + 16 * g) * SEQ for g in range(4)]
        for h in range(2):
            pltpu.sync_copy(
                idx_hbm.at[pl.ds(tile_idx0 + h * half, half)], idx_h
            )

            @pl.loop(0, SEQ)
            def _(s):
                for g in range(4):
                    v = plsc.load_gather(idx_h, [pat[g] + s])
                    idx_s[s, 64 * h + 16 * g:64 * h + 16 * g + 16] = v

        dla = lax.iota(jnp.int32, LANES)
        dt_vec = [(dla + 16 * dc) >> 3 for dc in range(4)]
        di_vec = [(dla + 16 * dc) & 7 for dc in range(4)]

        def fire_gather(s, j):
            pltpu.async_copy(table_hbm.at[idx_s.at[s]], rows[j], gsems[j])

        def drain_gather(j):
            pltpu.make_async_copy(
                table_hbm.at[idx_s.at[0]], rows[j], gsems[j]
            ).wait()

        def fire_write(s, j):
            pltpu.async_copy(
                stages[j].at[:, :, pl.ds(0, BT)],
                out_hbm.at[s, :, wid],
                wsems[j],
            )

        def drain_write(j):
            pltpu.make_async_copy(
                stages[j].at[:, :, pl.ds(0, BT)], out_hbm.at[0, :, 0],
                wsems[j],
            ).wait()

        def compute(s, j):
            rbuf = rows[j]
            sbuf = stages[j]
            pe_regs = [pe_v[s, 16 * dc:16 * dc + 16] for dc in range(4)]

            @plsc.parallel_loop(0, BT, 1, unroll=8)
            def _(b):
                bv = jnp.full((LANES,), b, jnp.int32)
                for dc in range(4):
                    v = rbuf[b, 16 * dc:16 * dc + 16]
                    plsc.store_scatter(
                        sbuf,
                        [dt_vec[dc], di_vec[dc], bv],
                        v * 8.0 + pe_regs[dc],
                    )

        def step(p, s, j):
            drain_gather(j)

            @pl.when(p > 0)
            def _():
                drain_write(j)

            compute(s, j)
            fire_write(s, j)

            @pl.when(p < SEQ // 2 - 1)
            def _():
                fire_gather(s + 2, j)

        fire_gather(0, 0)
        fire_gather(1, 1)

        @pl.loop(0, SEQ // 2)
        def _(p):
            step(p, 2 * p, 0)
            step(p, 2 * p + 1, 1)

        drain_write(0)
        drain_write(1)

    out5 = k(xi, w128, pe)
    out = out5.transpose(2, 4, 0, 1, 3).reshape(BATCH, SEQ, DIM)
    return out
